# Initial kernel scaffold; baseline (speedup 1.0000x reference)
import jax
import jax.numpy as jnp
from jax.experimental import pallas as pl


def kernel(x, Wg, W1, W2):
    raise NotImplementedError("write your pallas kernel here")


# speedup vs baseline: 3.7601x; 3.7601x over previous
import functools

import jax
import jax.numpy as jnp
from jax.experimental import pallas as pl
from jax.experimental.pallas import tpu as pltpu

_E = 8
_K = 2
_T = 4096
_D = 768
_H = _D // 2
_CAP = int(_T * _K / _E * 1.25)

_BT_R = 256
_BT_E = 512


def _router_body(x_ref, wg_ref, coef_ref, carry_ref):
    b = pl.program_id(0)

    @pl.when(b == 0)
    def _():
        carry_ref[...] = jnp.zeros_like(carry_ref)

    x = x_ref[...]
    wg = wg_ref[...]
    logits = jnp.dot(x, wg, preferred_element_type=jnp.float32)

    iota_e = jax.lax.broadcasted_iota(jnp.int32, logits.shape, 1)
    m1 = jnp.max(logits, axis=1, keepdims=True)
    i1 = jnp.min(jnp.where(logits == m1, iota_e, _E), axis=1, keepdims=True)
    sel1 = iota_e == i1
    l2 = jnp.where(sel1, -jnp.inf, logits)
    m2 = jnp.max(l2, axis=1, keepdims=True)
    i2 = jnp.min(jnp.where(l2 == m2, iota_e, _E), axis=1, keepdims=True)
    sel2 = iota_e == i2

    g1 = 1.0 / (1.0 + jnp.exp(m2 - m1))
    g2 = 1.0 - g1

    cnt = sel1.astype(jnp.float32) + sel2.astype(jnp.float32)
    ii = jax.lax.broadcasted_iota(jnp.int32, (_BT_R, _BT_R), 0)
    jj = jax.lax.broadcasted_iota(jnp.int32, (_BT_R, _BT_R), 1)
    lt = (jj < ii).astype(jnp.float32)
    cum = jnp.dot(lt, cnt, preferred_element_type=jnp.float32) + carry_ref[...]
    carry_ref[...] += jnp.sum(cnt, axis=0, keepdims=True)

    pos1 = jnp.sum(jnp.where(sel1, cum, 0.0), axis=1, keepdims=True)
    pos2 = jnp.sum(jnp.where(sel2, cum, 0.0), axis=1, keepdims=True)
    keep1 = (pos1 < _CAP).astype(jnp.float32)
    keep2 = (pos2 < _CAP).astype(jnp.float32)

    coef_ref[...] = (sel1.astype(jnp.float32) * (g1 * keep1)
                     + sel2.astype(jnp.float32) * (g2 * keep2))


def _expert_body(x_ref, w1_ref, w2_ref, coef_ref, out_ref):
    e = pl.program_id(1)
    x = x_ref[...]
    h = jnp.maximum(jnp.dot(x, w1_ref[0], preferred_element_type=jnp.float32), 0.0)
    y = jnp.maximum(jnp.dot(h, w2_ref[0], preferred_element_type=jnp.float32), 0.0)
    iota_e = jax.lax.broadcasted_iota(jnp.int32, (_BT_E, _E), 1)
    c = jnp.sum(jnp.where(iota_e == e, coef_ref[...], 0.0), axis=1, keepdims=True)

    @pl.when(e == 0)
    def _():
        out_ref[...] = x + c * y

    @pl.when(e != 0)
    def _():
        out_ref[...] += c * y


def kernel(x, Wg, W1, W2):
    coef = pl.pallas_call(
        _router_body,
        grid=(_T // _BT_R,),
        in_specs=[
            pl.BlockSpec((_BT_R, _D), lambda b: (b, 0)),
            pl.BlockSpec((_D, _E), lambda b: (0, 0)),
        ],
        out_specs=pl.BlockSpec((_BT_R, _E), lambda b: (b, 0)),
        out_shape=jax.ShapeDtypeStruct((_T, _E), jnp.float32),
        scratch_shapes=[pltpu.VMEM((1, _E), jnp.float32)],
    )(x, Wg)

    out = pl.pallas_call(
        _expert_body,
        grid=(_T // _BT_E, _E),
        in_specs=[
            pl.BlockSpec((_BT_E, _D), lambda t, e: (t, 0)),
            pl.BlockSpec((1, _D, _H), lambda t, e: (e, 0, 0)),
            pl.BlockSpec((1, _H, _D), lambda t, e: (e, 0, 0)),
            pl.BlockSpec((_BT_E, _E), lambda t, e: (t, 0)),
        ],
        out_specs=pl.BlockSpec((_BT_E, _D), lambda t, e: (t, 0)),
        out_shape=jax.ShapeDtypeStruct((_T, _D), jnp.float32),
    )(x, W1, W2, coef)
    return out
